# baseline (speedup 1.0000x reference)
import jax
import jax.numpy as jnp
from jax.experimental import pallas as pl


def _fused_body(sel_ref, items_ref, out_ref):
    s = sel_ref[...]
    m = jnp.max(s, axis=-1, keepdims=True)
    e = jnp.exp(s - m)
    z = jnp.sum(e, axis=-1, keepdims=True)
    acc = jnp.dot(e, items_ref[...], preferred_element_type=jnp.float32)
    out_ref[...] = acc / z


def kernel(selections, items):
    batch, n_items = selections.shape
    _, n_samples = items.shape
    tb = 512
    grid = (batch // tb,)
    return pl.pallas_call(
        _fused_body,
        grid=grid,
        in_specs=[
            pl.BlockSpec((tb, n_items), lambda i: (i, 0)),
            pl.BlockSpec((n_items, n_samples), lambda i: (0, 0)),
        ],
        out_specs=pl.BlockSpec((tb, n_samples), lambda i: (i, 0)),
        out_shape=jax.ShapeDtypeStruct((batch, n_samples), jnp.float32),
    )(selections, items)


# speedup vs baseline: 1.1015x; 1.1015x over previous
import jax
import jax.numpy as jnp
from jax.experimental import pallas as pl


def _fused_body(sel_ref, items_ref, out_ref):
    s = sel_ref[...]
    m = jnp.max(s, axis=-1, keepdims=True)
    e = jnp.exp(s - m)
    z = jnp.sum(e, axis=-1, keepdims=True)
    acc = jnp.dot(e, items_ref[...], preferred_element_type=jnp.float32)
    out_ref[...] = acc / z


def kernel(selections, items):
    batch, n_items = selections.shape
    _, n_samples = items.shape
    tb = 1024
    grid = (batch // tb,)
    return pl.pallas_call(
        _fused_body,
        grid=grid,
        in_specs=[
            pl.BlockSpec((tb, n_items), lambda i: (i, 0)),
            pl.BlockSpec((n_items, n_samples), lambda i: (0, 0)),
        ],
        out_specs=pl.BlockSpec((tb, n_samples), lambda i: (i, 0)),
        out_shape=jax.ShapeDtypeStruct((batch, n_samples), jnp.float32),
    )(selections, items)


# speedup vs baseline: 1.1520x; 1.0458x over previous
import jax
import jax.numpy as jnp
from jax.experimental import pallas as pl
from jax.experimental.pallas import tpu as pltpu

_TB = 512
_NBUF = 4


def _body(sel_hbm, items_ref, out_ref, buf, sems):
    n_chunks = out_ref.shape[0] // _TB
    items = items_ref[...]

    def start_copy(chunk, slot):
        pltpu.make_async_copy(
            sel_hbm.at[pl.ds(chunk * _TB, _TB), :],
            buf.at[slot],
            sems.at[slot],
        ).start()

    for k in range(_NBUF):
        start_copy(k, k)

    def step(i, _):
        slot = jax.lax.rem(i, _NBUF)
        pltpu.make_async_copy(
            sel_hbm.at[pl.ds(i * _TB, _TB), :],
            buf.at[slot],
            sems.at[slot],
        ).wait()
        s = buf[slot]
        m = jnp.max(s, axis=-1, keepdims=True)
        e = jnp.exp(s - m)
        z = jnp.sum(e, axis=-1, keepdims=True)
        acc = jnp.dot(e, items, preferred_element_type=jnp.float32)
        out_ref[pl.ds(i * _TB, _TB), :] = acc / z

        @pl.when(i + _NBUF < n_chunks)
        def _():
            start_copy(i + _NBUF, slot)

        return 0

    jax.lax.fori_loop(0, n_chunks, step, 0)


def kernel(selections, items):
    batch, n_items = selections.shape
    _, n_samples = items.shape
    return pl.pallas_call(
        _body,
        in_specs=[
            pl.BlockSpec(memory_space=pltpu.MemorySpace.HBM),
            pl.BlockSpec(memory_space=pltpu.MemorySpace.VMEM),
        ],
        out_specs=pl.BlockSpec(memory_space=pltpu.MemorySpace.VMEM),
        out_shape=jax.ShapeDtypeStruct((batch, n_samples), jnp.float32),
        scratch_shapes=[
            pltpu.VMEM((_NBUF, _TB, n_items), jnp.float32),
            pltpu.SemaphoreType.DMA((_NBUF,)),
        ],
    )(selections, items)
